# baseline (speedup 1.0000x reference)
import functools

import jax
import jax.numpy as jnp
from jax import lax
from jax.experimental import pallas as pl
from jax.experimental.pallas import tpu as pltpu
from jax.experimental.pallas import tpu_sc as plsc

N_VOCAB = 49408
N_EMBD = 768
N_TOKEN = 77
BATCH = 1024

NC = 2
NS = 16
NW = NC * NS

T_PAD = 80
FLAT = BATCH * T_PAD
ROWS_W = FLAT // NW
CHUNK = 64
NCH = ROWS_W // CHUNK

BB = 8

_mesh = plsc.VectorSubcoreMesh(
    core_axis_name="c", subcore_axis_name="s", num_cores=NC, num_subcores=NS
)


@functools.partial(
    pl.kernel,
    out_type=jax.ShapeDtypeStruct((FLAT, N_EMBD), jnp.float32),
    mesh=_mesh,
    scratch_types=[
        pltpu.VMEM((ROWS_W,), jnp.int32),
        pltpu.VMEM((2, CHUNK, N_EMBD), jnp.float32),
        pltpu.SemaphoreType.DMA,
        pltpu.SemaphoreType.DMA,
        pltpu.SemaphoreType.DMA,
        pltpu.SemaphoreType.DMA,
    ],
)
def _gather_rows(tok_hbm, tab_hbm, out_hbm, idx_v, rows_v, g0, g1, s0, s1):
    gsem = (g0, g1)
    ssem = (s0, s1)
    wid = lax.axis_index("s") * NC + lax.axis_index("c")
    base = wid * ROWS_W

    pltpu.sync_copy(tok_hbm.at[pl.ds(base, ROWS_W)], idx_v)

    def gather_desc(c, b):
        return pltpu.make_async_copy(
            tab_hbm.at[idx_v.at[pl.ds(c * CHUNK, CHUNK)]], rows_v.at[b], gsem[b]
        )

    def scatter_desc(c, b):
        return pltpu.make_async_copy(
            rows_v.at[b], out_hbm.at[pl.ds(base + c * CHUNK, CHUNK)], ssem[b]
        )

    gather_desc(0, 0).start()

    def pair_body(jj, carry):
        for b in range(2):
            j = jj * 2 + b
            bn = 1 - b

            @pl.when(j >= 1)
            def _():
                scatter_desc(j - 1, bn).wait()

            @pl.when(j + 1 < NCH)
            def _():
                gather_desc(j + 1, bn).start()

            gather_desc(j, b).wait()
            scatter_desc(j, b).start()
        return carry

    lax.fori_loop(0, NCH // 2, pair_body, 0)

    scatter_desc(NCH - 1, (NCH - 1) % 2).wait()


def _add_pos_body(tmp_ref, pos_ref, out_ref):
    x = tmp_ref[...].reshape(BB, T_PAD, N_EMBD)
    out_ref[...] = x[:, :N_TOKEN, :] + pos_ref[...][None, :, :]


_add_pos = pl.pallas_call(
    _add_pos_body,
    grid=(BATCH // BB,),
    in_specs=[
        pl.BlockSpec((BB * T_PAD, N_EMBD), lambda g: (g, 0)),
        pl.BlockSpec((N_TOKEN, N_EMBD), lambda g: (0, 0)),
    ],
    out_specs=pl.BlockSpec((BB, N_TOKEN, N_EMBD), lambda g: (g, 0, 0)),
    out_shape=jax.ShapeDtypeStruct((BATCH, N_TOKEN, N_EMBD), jnp.float32),
)


def kernel(tokens, token_embedding, position_value):
    tok = jnp.pad(tokens.astype(jnp.int32), ((0, 0), (0, T_PAD - N_TOKEN)))
    tmp = _gather_rows(tok.reshape(-1), token_embedding)
    return _add_pos(tmp, position_value)
